# Initial kernel scaffold; baseline (speedup 1.0000x reference)
import functools

import jax
import jax.numpy as jnp
from jax.experimental import pallas as pl
from jax.experimental.pallas import tpu as pltpu

B, S, D, V = 1, 2048, 1024, 50258
L, H, FF = 2, 16, 4096
NPEERS, TOPK = 64, 8
HD = D // H

BM = 256
BQ = 512
BV = 2048
NV = (V + BV - 1) // BV


def _routing_body(pw_ref, act_ref, noise_ref, out_ref):
    pw = pw_ref[0, :]
    af = act_ref[0, :]
    noise = noise_ref[0, :]
    n = jnp.sum(af)
    mean = jnp.sum(pw * af) / n
    var = jnp.sum(af * (pw - mean) ** 2) / jnp.maximum(n - 1.0, 1.0)
    std = jnp.sqrt(var)
    scores = jnp.where(af > 0.5, pw + noise * (std + 1e-7), -1e9)
    s_i = scores.reshape(NPEERS, 1)
    s_j = scores.reshape(1, NPEERS)
    ii = jax.lax.broadcasted_iota(jnp.int32, (NPEERS, NPEERS), 0)
    jj = jax.lax.broadcasted_iota(jnp.int32, (NPEERS, NPEERS), 1)
    gt = (s_j > s_i).astype(jnp.int32)
    eq = ((s_j == s_i) & (jj < ii)).astype(jnp.int32)
    rank = jnp.sum(gt + eq, axis=1)
    e = jax.lax.broadcasted_iota(jnp.int32, (TOPK, NPEERS), 0)
    sel = (rank.reshape(1, NPEERS) == e).astype(jnp.float32)
    top_vals = jnp.sum(sel * scores.reshape(1, NPEERS), axis=1)
    m = jnp.max(top_vals)
    ex = jnp.exp(top_vals - m)
    out_ref[0, :] = ex / jnp.sum(ex)


def _routing(pw, af, noise):
    return pl.pallas_call(
        _routing_body,
        out_shape=jax.ShapeDtypeStruct((1, TOPK), jnp.float32),
        in_specs=[pl.BlockSpec((1, NPEERS), lambda: (0, 0))] * 3,
        out_specs=pl.BlockSpec((1, TOPK), lambda: (0, 0)),
    )(pw.reshape(1, NPEERS), af.reshape(1, NPEERS), noise.reshape(1, NPEERS))


def _combine_body(w_ref, resp_ref, out_ref):
    acc = w_ref[0, 0] * resp_ref[0]
    for e in range(1, TOPK):
        acc = acc + w_ref[0, e] * resp_ref[e]
    out_ref[...] = acc


def _combine(jw, resp):
    return pl.pallas_call(
        _combine_body,
        grid=(S // BM,),
        in_specs=[
            pl.BlockSpec((1, TOPK), lambda s: (0, 0)),
            pl.BlockSpec((TOPK, BM, D), lambda s: (0, s, 0)),
        ],
        out_specs=pl.BlockSpec((BM, D), lambda s: (s, 0)),
        out_shape=jax.ShapeDtypeStruct((S, D), jnp.float32),
    )(jw, resp)


def _mm_bias_body(a_ref, w_ref, b_ref, out_ref, *, relu):
    a = a_ref[...].astype(jnp.bfloat16)
    w = w_ref[...].astype(jnp.bfloat16)
    acc = jnp.dot(a, w, preferred_element_type=jnp.float32) + b_ref[...]
    if relu:
        acc = jnp.maximum(acc, 0.0)
    out_ref[...] = acc.astype(out_ref.dtype)


def _mm_bias(a, w, b, relu=False, out_dtype=jnp.bfloat16):
    m, k = a.shape
    n = w.shape[1]
    return pl.pallas_call(
        functools.partial(_mm_bias_body, relu=relu),
        grid=(m // BM,),
        in_specs=[
            pl.BlockSpec((BM, k), lambda i: (i, 0)),
            pl.BlockSpec((k, n), lambda i: (0, 0)),
            pl.BlockSpec((1, n), lambda i: (0, 0)),
        ],
        out_specs=pl.BlockSpec((BM, n), lambda i: (i, 0)),
        out_shape=jax.ShapeDtypeStruct((m, n), out_dtype),
    )(a, w, b.reshape(1, n))


def _attn_body(q_ref, k_ref, v_ref, out_ref):
    q = q_ref[...].astype(jnp.bfloat16)
    k = k_ref[...].astype(jnp.bfloat16)
    s = jax.lax.dot_general(q, k, (((1,), (1,)), ((), ())),
                            preferred_element_type=jnp.float32)
    s = s * (1.0 / (HD ** 0.5))
    m = jnp.max(s, axis=1, keepdims=True)
    p = jnp.exp(s - m)
    p = (p / jnp.sum(p, axis=1, keepdims=True)).astype(jnp.bfloat16)
    v = v_ref[...].astype(jnp.bfloat16)
    out_ref[...] = jnp.dot(p, v, preferred_element_type=jnp.float32
                           ).astype(jnp.bfloat16)


def _attention(qkv):
    return pl.pallas_call(
        _attn_body,
        grid=(H, S // BQ),
        in_specs=[
            pl.BlockSpec((BQ, HD), lambda h, sq: (sq, h)),
            pl.BlockSpec((S, HD), lambda h, sq: (0, H + h)),
            pl.BlockSpec((S, HD), lambda h, sq: (0, 2 * H + h)),
        ],
        out_specs=pl.BlockSpec((BQ, HD), lambda h, sq: (sq, h)),
        out_shape=jax.ShapeDtypeStruct((S, D), jnp.bfloat16),
    )(qkv, qkv, qkv)


def _mm_res_ln_body(a_ref, w_ref, b_ref, res_ref, g_ref, beta_ref, out_ref):
    a = a_ref[...].astype(jnp.bfloat16)
    w = w_ref[...].astype(jnp.bfloat16)
    h = jnp.dot(a, w, preferred_element_type=jnp.float32)
    h = h + b_ref[...] + res_ref[...]
    mu = jnp.mean(h, axis=1, keepdims=True)
    var = jnp.mean((h - mu) ** 2, axis=1, keepdims=True)
    out_ref[...] = (h - mu) / jnp.sqrt(var + 1e-5) * g_ref[...] + beta_ref[...]


def _mm_res_ln(a, w, b, res, g, beta):
    m, k = a.shape
    n = w.shape[1]
    return pl.pallas_call(
        _mm_res_ln_body,
        grid=(m // BM,),
        in_specs=[
            pl.BlockSpec((BM, k), lambda i: (i, 0)),
            pl.BlockSpec((k, n), lambda i: (0, 0)),
            pl.BlockSpec((1, n), lambda i: (0, 0)),
            pl.BlockSpec((BM, n), lambda i: (i, 0)),
            pl.BlockSpec((1, n), lambda i: (0, 0)),
            pl.BlockSpec((1, n), lambda i: (0, 0)),
        ],
        out_specs=pl.BlockSpec((BM, n), lambda i: (i, 0)),
        out_shape=jax.ShapeDtypeStruct((m, n), jnp.float32),
    )(a, w, b.reshape(1, n), res, g.reshape(1, n), beta.reshape(1, n))


def _dec_body(x_ref, w_ref, lbl_ref, out_ref, loss_ref,
              m_scr, s_scr, ll_scr, acc_scr):
    v = pl.program_id(0)
    si = pl.program_id(1)
    rows = pl.ds(si * BM, BM)
    x = x_ref[rows, :].astype(jnp.bfloat16)
    w = w_ref[...].astype(jnp.bfloat16)
    t = jnp.dot(x, w, preferred_element_type=jnp.float32)
    out_ref[...] = t

    cols = v * BV + jax.lax.broadcasted_iota(jnp.int32, (BM, BV), 1)
    valid = cols < V
    tm = jnp.where(valid, t, -1e30)

    first = v == 0
    m_old = jnp.where(first, jnp.full((BM, 1), -1e30, jnp.float32),
                      m_scr[rows, :])
    s_old = jnp.where(first, jnp.zeros((BM, 1), jnp.float32), s_scr[rows, :])
    ll_old = jnp.where(first, jnp.zeros((BM, 1), jnp.float32), ll_scr[rows, :])

    m_new = jnp.maximum(m_old, jnp.max(tm, axis=1, keepdims=True))
    s_new = s_old * jnp.exp(m_old - m_new) + jnp.sum(
        jnp.exp(tm - m_new), axis=1, keepdims=True)
    lbl = lbl_ref[...]
    match = (cols == lbl) & valid
    ll_new = ll_old + jnp.sum(jnp.where(match, t, 0.0), axis=1, keepdims=True)

    m_scr[rows, :] = m_new
    s_scr[rows, :] = s_new
    ll_scr[rows, :] = ll_new

    @pl.when(v == NV - 1)
    def _():
        nll = m_new + jnp.log(s_new) - ll_new
        rg = si * BM + jax.lax.broadcasted_iota(jnp.int32, (BM, 1), 0)
        blk = jnp.sum(jnp.where(rg < S - 1, nll, 0.0))
        prev = jnp.where(si == 0, 0.0, acc_scr[0, 0])
        tot = prev + blk
        acc_scr[0, 0] = tot

        @pl.when(si == S // BM - 1)
        def _():
            loss_ref[0, 0] = tot / (S - 1.0)


def _decoder_loss(x, w_dec, labels):
    return pl.pallas_call(
        _dec_body,
        grid=(NV, S // BM),
        in_specs=[
            pl.BlockSpec((S, D), lambda v, s: (0, 0)),
            pl.BlockSpec((D, BV), lambda v, s: (0, v)),
            pl.BlockSpec((BM, 1), lambda v, s: (s, 0)),
        ],
        out_specs=[
            pl.BlockSpec((BM, BV), lambda v, s: (s, v)),
            pl.BlockSpec((1, 1), lambda v, s: (0, 0)),
        ],
        out_shape=[
            jax.ShapeDtypeStruct((S, V), jnp.float32),
            jax.ShapeDtypeStruct((1, 1), jnp.float32),
        ],
        scratch_shapes=[
            pltpu.VMEM((S, 1), jnp.float32),
            pltpu.VMEM((S, 1), jnp.float32),
            pltpu.VMEM((S, 1), jnp.float32),
            pltpu.SMEM((1, 1), jnp.float32),
        ],
    )(x, w_dec, labels)


def kernel(inputs, active, responses, peer_weights, Wqkv, bqkv, Wo, bo,
           W1, b1, W2, b2, ln1_g, ln1_b, ln2_g, ln2_b, W_dec):
    noise = jax.random.normal(jax.random.key(42), peer_weights.shape,
                              dtype=jnp.float32)
    af = active.astype(jnp.float32)
    jw = _routing(peer_weights, af, noise)
    x = _combine(jw, responses.reshape(TOPK, S, D))
    for l in range(L):
        qkv = _mm_bias(x, Wqkv[l], bqkv[l])
        o = _attention(qkv)
        x = _mm_res_ln(o, Wo[l], bo[l], x, ln1_g[l], ln1_b[l])
        h = _mm_bias(x, W1[l], b1[l], relu=True)
        x = _mm_res_ln(h, W2[l], b2[l], x, ln2_g[l], ln2_b[l])
    labels = jnp.concatenate(
        [inputs[0, 1:], jnp.zeros((1,), inputs.dtype)]).astype(jnp.int32)
    logits, loss = _decoder_loss(x, W_dec, labels.reshape(S, 1))
    return loss.reshape(()), logits.reshape(B, S, V)


# speedup vs baseline: 1.1842x; 1.1842x over previous
import functools

import jax
import jax.numpy as jnp
from jax.experimental import pallas as pl
from jax.experimental.pallas import tpu as pltpu

B, S, D, V = 1, 2048, 1024, 50258
L, H, FF = 2, 16, 4096
NPEERS, TOPK = 64, 8
HD = D // H

BM = 256
BQ = 512
BV = 2048
NV = (V + BV - 1) // BV


def _routing_body(pw_ref, act_ref, noise_ref, out_ref):
    pw = pw_ref[0, :]
    af = act_ref[0, :]
    noise = noise_ref[0, :]
    n = jnp.sum(af)
    mean = jnp.sum(pw * af) / n
    var = jnp.sum(af * (pw - mean) ** 2) / jnp.maximum(n - 1.0, 1.0)
    std = jnp.sqrt(var)
    scores = jnp.where(af > 0.5, pw + noise * (std + 1e-7), -1e9)
    s_i = scores.reshape(NPEERS, 1)
    s_j = scores.reshape(1, NPEERS)
    ii = jax.lax.broadcasted_iota(jnp.int32, (NPEERS, NPEERS), 0)
    jj = jax.lax.broadcasted_iota(jnp.int32, (NPEERS, NPEERS), 1)
    gt = (s_j > s_i).astype(jnp.int32)
    eq = ((s_j == s_i) & (jj < ii)).astype(jnp.int32)
    rank = jnp.sum(gt + eq, axis=1)
    e = jax.lax.broadcasted_iota(jnp.int32, (TOPK, NPEERS), 0)
    sel = (rank.reshape(1, NPEERS) == e).astype(jnp.float32)
    top_vals = jnp.sum(sel * scores.reshape(1, NPEERS), axis=1)
    m = jnp.max(top_vals)
    ex = jnp.exp(top_vals - m)
    out_ref[0, :] = ex / jnp.sum(ex)


def _routing(pw, af, noise):
    return pl.pallas_call(
        _routing_body,
        out_shape=jax.ShapeDtypeStruct((1, TOPK), jnp.float32),
        in_specs=[pl.BlockSpec((1, NPEERS), lambda: (0, 0))] * 3,
        out_specs=pl.BlockSpec((1, TOPK), lambda: (0, 0)),
    )(pw.reshape(1, NPEERS), af.reshape(1, NPEERS), noise.reshape(1, NPEERS))


def _combine_body(w_ref, resp_ref, out_ref):
    acc = w_ref[0, 0] * resp_ref[0]
    for e in range(1, TOPK):
        acc = acc + w_ref[0, e] * resp_ref[e]
    out_ref[...] = acc


def _combine(jw, resp):
    return pl.pallas_call(
        _combine_body,
        grid=(S // BM,),
        in_specs=[
            pl.BlockSpec((1, TOPK), lambda s: (0, 0)),
            pl.BlockSpec((TOPK, BM, D), lambda s: (0, s, 0)),
        ],
        out_specs=pl.BlockSpec((BM, D), lambda s: (s, 0)),
        out_shape=jax.ShapeDtypeStruct((S, D), jnp.float32),
    )(jw, resp)


def _mm_bias_body(a_ref, w_ref, b_ref, out_ref, *, relu):
    a = a_ref[...].astype(jnp.bfloat16)
    w = w_ref[...].astype(jnp.bfloat16)
    acc = jnp.dot(a, w, preferred_element_type=jnp.float32) + b_ref[...]
    if relu:
        acc = jnp.maximum(acc, 0.0)
    out_ref[...] = acc.astype(out_ref.dtype)


def _mm_bias(a, w, b, relu=False, out_dtype=jnp.bfloat16):
    m, k = a.shape
    n = w.shape[1]
    return pl.pallas_call(
        functools.partial(_mm_bias_body, relu=relu),
        grid=(m // BM,),
        in_specs=[
            pl.BlockSpec((BM, k), lambda i: (i, 0)),
            pl.BlockSpec((k, n), lambda i: (0, 0)),
            pl.BlockSpec((1, n), lambda i: (0, 0)),
        ],
        out_specs=pl.BlockSpec((BM, n), lambda i: (i, 0)),
        out_shape=jax.ShapeDtypeStruct((m, n), out_dtype),
    )(a, w, b.reshape(1, n))


def _attn_body(q_ref, k_ref, v_ref, out_ref):
    q2 = q_ref[...].astype(jnp.bfloat16)
    k2 = k_ref[...].astype(jnp.bfloat16)
    v2 = v_ref[...].astype(jnp.bfloat16)
    outs = []
    for i in range(2):
        q = q2[:, i * HD:(i + 1) * HD]
        k = k2[:, i * HD:(i + 1) * HD]
        v = v2[:, i * HD:(i + 1) * HD]
        s = jax.lax.dot_general(q, k, (((1,), (1,)), ((), ())),
                                preferred_element_type=jnp.float32)
        s = s * (1.0 / (HD ** 0.5))
        m = jnp.max(s, axis=1, keepdims=True)
        p = jnp.exp(s - m)
        p = (p / jnp.sum(p, axis=1, keepdims=True)).astype(jnp.bfloat16)
        outs.append(jnp.dot(p, v, preferred_element_type=jnp.float32))
    out_ref[...] = jnp.concatenate(outs, axis=1).astype(jnp.bfloat16)


def _attention(qkv):
    HP = H // 2
    return pl.pallas_call(
        _attn_body,
        grid=(HP, S // BQ),
        in_specs=[
            pl.BlockSpec((BQ, 2 * HD), lambda h, sq: (sq, h)),
            pl.BlockSpec((S, 2 * HD), lambda h, sq: (0, HP + h)),
            pl.BlockSpec((S, 2 * HD), lambda h, sq: (0, 2 * HP + h)),
        ],
        out_specs=pl.BlockSpec((BQ, 2 * HD), lambda h, sq: (sq, h)),
        out_shape=jax.ShapeDtypeStruct((S, D), jnp.bfloat16),
    )(qkv, qkv, qkv)


def _mm_res_ln_body(a_ref, w_ref, b_ref, res_ref, g_ref, beta_ref, out_ref):
    a = a_ref[...].astype(jnp.bfloat16)
    w = w_ref[...].astype(jnp.bfloat16)
    h = jnp.dot(a, w, preferred_element_type=jnp.float32)
    h = h + b_ref[...] + res_ref[...]
    mu = jnp.mean(h, axis=1, keepdims=True)
    var = jnp.mean((h - mu) ** 2, axis=1, keepdims=True)
    out_ref[...] = (h - mu) / jnp.sqrt(var + 1e-5) * g_ref[...] + beta_ref[...]


def _mm_res_ln(a, w, b, res, g, beta):
    m, k = a.shape
    n = w.shape[1]
    return pl.pallas_call(
        _mm_res_ln_body,
        grid=(m // BM,),
        in_specs=[
            pl.BlockSpec((BM, k), lambda i: (i, 0)),
            pl.BlockSpec((k, n), lambda i: (0, 0)),
            pl.BlockSpec((1, n), lambda i: (0, 0)),
            pl.BlockSpec((BM, n), lambda i: (i, 0)),
            pl.BlockSpec((1, n), lambda i: (0, 0)),
            pl.BlockSpec((1, n), lambda i: (0, 0)),
        ],
        out_specs=pl.BlockSpec((BM, n), lambda i: (i, 0)),
        out_shape=jax.ShapeDtypeStruct((m, n), jnp.float32),
    )(a, w, b.reshape(1, n), res, g.reshape(1, n), beta.reshape(1, n))


def _dec_body(x_ref, w_ref, lbl_ref, out_ref, loss_ref,
              m_scr, s_scr, ll_scr, acc_scr):
    v = pl.program_id(0)
    si = pl.program_id(1)
    rows = pl.ds(si * BM, BM)
    x = x_ref[rows, :].astype(jnp.bfloat16)
    w = w_ref[...].astype(jnp.bfloat16)
    t = jnp.dot(x, w, preferred_element_type=jnp.float32)
    out_ref[...] = t

    cols = v * BV + jax.lax.broadcasted_iota(jnp.int32, (BM, BV), 1)
    valid = cols < V
    tm = jnp.where(valid, t, -1e30)

    first = v == 0
    m_old = jnp.where(first, jnp.full((BM, 1), -1e30, jnp.float32),
                      m_scr[rows, :])
    s_old = jnp.where(first, jnp.zeros((BM, 1), jnp.float32), s_scr[rows, :])
    ll_old = jnp.where(first, jnp.zeros((BM, 1), jnp.float32), ll_scr[rows, :])

    m_new = jnp.maximum(m_old, jnp.max(tm, axis=1, keepdims=True))
    s_new = s_old * jnp.exp(m_old - m_new) + jnp.sum(
        jnp.exp(tm - m_new), axis=1, keepdims=True)
    lbl = lbl_ref[...]
    match = (cols == lbl) & valid
    ll_new = ll_old + jnp.sum(jnp.where(match, t, 0.0), axis=1, keepdims=True)

    m_scr[rows, :] = m_new
    s_scr[rows, :] = s_new
    ll_scr[rows, :] = ll_new

    @pl.when(v == NV - 1)
    def _():
        nll = m_new + jnp.log(s_new) - ll_new
        rg = si * BM + jax.lax.broadcasted_iota(jnp.int32, (BM, 1), 0)
        blk = jnp.sum(jnp.where(rg < S - 1, nll, 0.0))
        prev = jnp.where(si == 0, 0.0, acc_scr[0, 0])
        tot = prev + blk
        acc_scr[0, 0] = tot

        @pl.when(si == S // BM - 1)
        def _():
            loss_ref[...] = (tot / (S - 1.0)).reshape(1, 1)


def _decoder_loss(x, w_dec, labels):
    return pl.pallas_call(
        _dec_body,
        grid=(NV, S // BM),
        in_specs=[
            pl.BlockSpec((S, D), lambda v, s: (0, 0)),
            pl.BlockSpec((D, BV), lambda v, s: (0, v)),
            pl.BlockSpec((BM, 1), lambda v, s: (s, 0)),
        ],
        out_specs=[
            pl.BlockSpec((BM, BV), lambda v, s: (s, v)),
            pl.BlockSpec((1, 1), lambda v, s: (0, 0)),
        ],
        out_shape=[
            jax.ShapeDtypeStruct((S, V), jnp.float32),
            jax.ShapeDtypeStruct((1, 1), jnp.float32),
        ],
        scratch_shapes=[
            pltpu.VMEM((S, 1), jnp.float32),
            pltpu.VMEM((S, 1), jnp.float32),
            pltpu.VMEM((S, 1), jnp.float32),
            pltpu.SMEM((1, 1), jnp.float32),
        ],
    )(x, w_dec, labels)


def kernel(inputs, active, responses, peer_weights, Wqkv, bqkv, Wo, bo,
           W1, b1, W2, b2, ln1_g, ln1_b, ln2_g, ln2_b, W_dec):
    noise = jax.random.normal(jax.random.key(42), peer_weights.shape,
                              dtype=jnp.float32)
    af = active.astype(jnp.float32)
    jw = _routing(peer_weights, af, noise)
    x = _combine(jw, responses.reshape(TOPK, S, D))
    for l in range(L):
        qkv = _mm_bias(x, Wqkv[l], bqkv[l])
        o = _attention(qkv)
        x = _mm_res_ln(o, Wo[l], bo[l], x, ln1_g[l], ln1_b[l])
        h = _mm_bias(x, W1[l], b1[l], relu=True)
        x = _mm_res_ln(h, W2[l], b2[l], x, ln2_g[l], ln2_b[l])
    labels = jnp.concatenate(
        [inputs[0, 1:], jnp.zeros((1,), inputs.dtype)]).astype(jnp.int32)
    logits, loss = _decoder_loss(x, W_dec, labels.reshape(S, 1))
    return loss.reshape(()), logits.reshape(B, S, V)


# speedup vs baseline: 1.1849x; 1.0007x over previous
import functools

import jax
import jax.numpy as jnp
from jax.experimental import pallas as pl
from jax.experimental.pallas import tpu as pltpu

B, S, D, V = 1, 2048, 1024, 50258
L, H, FF = 2, 16, 4096
NPEERS, TOPK = 64, 8
HD = D // H

BM = 256
BQ = 512
BV = 2048
NV = (V + BV - 1) // BV


def _routing_body(pw_ref, act_ref, noise_ref, out_ref):
    pw = pw_ref[0, :]
    af = act_ref[0, :]
    noise = noise_ref[0, :]
    n = jnp.sum(af)
    mean = jnp.sum(pw * af) / n
    var = jnp.sum(af * (pw - mean) ** 2) / jnp.maximum(n - 1.0, 1.0)
    std = jnp.sqrt(var)
    scores = jnp.where(af > 0.5, pw + noise * (std + 1e-7), -1e9)
    s_i = scores.reshape(NPEERS, 1)
    s_j = scores.reshape(1, NPEERS)
    ii = jax.lax.broadcasted_iota(jnp.int32, (NPEERS, NPEERS), 0)
    jj = jax.lax.broadcasted_iota(jnp.int32, (NPEERS, NPEERS), 1)
    gt = (s_j > s_i).astype(jnp.int32)
    eq = ((s_j == s_i) & (jj < ii)).astype(jnp.int32)
    rank = jnp.sum(gt + eq, axis=1)
    e = jax.lax.broadcasted_iota(jnp.int32, (TOPK, NPEERS), 0)
    sel = (rank.reshape(1, NPEERS) == e).astype(jnp.float32)
    top_vals = jnp.sum(sel * scores.reshape(1, NPEERS), axis=1)
    m = jnp.max(top_vals)
    ex = jnp.exp(top_vals - m)
    out_ref[0, :] = ex / jnp.sum(ex)


def _routing(pw, af, noise):
    return pl.pallas_call(
        _routing_body,
        out_shape=jax.ShapeDtypeStruct((1, TOPK), jnp.float32),
        in_specs=[pl.BlockSpec((1, NPEERS), lambda: (0, 0))] * 3,
        out_specs=pl.BlockSpec((1, TOPK), lambda: (0, 0)),
    )(pw.reshape(1, NPEERS), af.reshape(1, NPEERS), noise.reshape(1, NPEERS))


def _combine_body(w_ref, resp_ref, out_ref):
    acc = w_ref[0, 0] * resp_ref[0]
    for e in range(1, TOPK):
        acc = acc + w_ref[0, e] * resp_ref[e]
    out_ref[...] = acc


def _combine(jw, resp):
    return pl.pallas_call(
        _combine_body,
        grid=(S // BM,),
        in_specs=[
            pl.BlockSpec((1, TOPK), lambda s: (0, 0)),
            pl.BlockSpec((TOPK, BM, D), lambda s: (0, s, 0)),
        ],
        out_specs=pl.BlockSpec((BM, D), lambda s: (s, 0)),
        out_shape=jax.ShapeDtypeStruct((S, D), jnp.float32),
    )(jw, resp)


def _mm_bias_body(a_ref, w_ref, b_ref, out_ref, *, relu):
    a = a_ref[...].astype(jnp.bfloat16)
    w = w_ref[...].astype(jnp.bfloat16)
    acc = jnp.dot(a, w, preferred_element_type=jnp.float32) + b_ref[...]
    if relu:
        acc = jnp.maximum(acc, 0.0)
    out_ref[...] = acc.astype(out_ref.dtype)


def _mm_qkv_body(a_ref, w_ref, b_ref, q_ref, k_ref, v_ref):
    a = a_ref[...].astype(jnp.bfloat16)
    w = w_ref[...].astype(jnp.bfloat16)
    acc = jnp.dot(a, w, preferred_element_type=jnp.float32) + b_ref[...]
    acc = acc.astype(jnp.bfloat16)
    q_ref[...] = acc[:, :D]
    k_ref[...] = acc[:, D:2 * D]
    v_ref[...] = acc[:, 2 * D:]


def _mm_qkv(a, w, b):
    m, k = a.shape
    n = w.shape[1]
    return pl.pallas_call(
        _mm_qkv_body,
        grid=(m // BM,),
        in_specs=[
            pl.BlockSpec((BM, k), lambda i: (i, 0)),
            pl.BlockSpec((k, n), lambda i: (0, 0)),
            pl.BlockSpec((1, n), lambda i: (0, 0)),
        ],
        out_specs=[pl.BlockSpec((BM, D), lambda i: (i, 0))] * 3,
        out_shape=[jax.ShapeDtypeStruct((m, D), jnp.bfloat16)] * 3,
    )(a, w, b.reshape(1, n))


def _mm_bias(a, w, b, relu=False, out_dtype=jnp.bfloat16):
    m, k = a.shape
    n = w.shape[1]
    return pl.pallas_call(
        functools.partial(_mm_bias_body, relu=relu),
        grid=(m // BM,),
        in_specs=[
            pl.BlockSpec((BM, k), lambda i: (i, 0)),
            pl.BlockSpec((k, n), lambda i: (0, 0)),
            pl.BlockSpec((1, n), lambda i: (0, 0)),
        ],
        out_specs=pl.BlockSpec((BM, n), lambda i: (i, 0)),
        out_shape=jax.ShapeDtypeStruct((m, n), out_dtype),
    )(a, w, b.reshape(1, n))


def _attn_body(q_ref, k_ref, v_ref, out_ref):
    q2 = q_ref[...]
    k2 = k_ref[...]
    v2 = v_ref[...]
    outs = []
    for i in range(2):
        q = q2[:, i * HD:(i + 1) * HD]
        k = k2[:, i * HD:(i + 1) * HD]
        v = v2[:, i * HD:(i + 1) * HD]
        s = jax.lax.dot_general(q, k, (((1,), (1,)), ((), ())),
                                preferred_element_type=jnp.float32)
        s = s * (1.0 / (HD ** 0.5))
        m = jnp.max(s, axis=1, keepdims=True)
        p = jnp.exp(s - m)
        p = (p / jnp.sum(p, axis=1, keepdims=True)).astype(jnp.bfloat16)
        outs.append(jnp.dot(p, v, preferred_element_type=jnp.float32))
    out_ref[...] = jnp.concatenate(outs, axis=1).astype(jnp.bfloat16)


def _attention(q, k, v):
    return pl.pallas_call(
        _attn_body,
        grid=(H // 2, S // BQ),
        in_specs=[
            pl.BlockSpec((BQ, 2 * HD), lambda h, sq: (sq, h)),
            pl.BlockSpec((S, 2 * HD), lambda h, sq: (0, h)),
            pl.BlockSpec((S, 2 * HD), lambda h, sq: (0, h)),
        ],
        out_specs=pl.BlockSpec((BQ, 2 * HD), lambda h, sq: (sq, h)),
        out_shape=jax.ShapeDtypeStruct((S, D), jnp.bfloat16),
    )(q, k, v)


def _mm_res_ln_body(a_ref, w_ref, b_ref, res_ref, g_ref, beta_ref, out_ref):
    a = a_ref[...].astype(jnp.bfloat16)
    w = w_ref[...].astype(jnp.bfloat16)
    h = jnp.dot(a, w, preferred_element_type=jnp.float32)
    h = h + b_ref[...] + res_ref[...]
    mu = jnp.mean(h, axis=1, keepdims=True)
    var = jnp.mean((h - mu) ** 2, axis=1, keepdims=True)
    out_ref[...] = (h - mu) / jnp.sqrt(var + 1e-5) * g_ref[...] + beta_ref[...]


def _mm_res_ln(a, w, b, res, g, beta):
    m, k = a.shape
    n = w.shape[1]
    return pl.pallas_call(
        _mm_res_ln_body,
        grid=(m // BM,),
        in_specs=[
            pl.BlockSpec((BM, k), lambda i: (i, 0)),
            pl.BlockSpec((k, n), lambda i: (0, 0)),
            pl.BlockSpec((1, n), lambda i: (0, 0)),
            pl.BlockSpec((BM, n), lambda i: (i, 0)),
            pl.BlockSpec((1, n), lambda i: (0, 0)),
            pl.BlockSpec((1, n), lambda i: (0, 0)),
        ],
        out_specs=pl.BlockSpec((BM, n), lambda i: (i, 0)),
        out_shape=jax.ShapeDtypeStruct((m, n), jnp.float32),
    )(a, w, b.reshape(1, n), res, g.reshape(1, n), beta.reshape(1, n))


def _dec_body(x_ref, w_ref, lbl_ref, out_ref, loss_ref,
              m_scr, s_scr, ll_scr, acc_scr):
    v = pl.program_id(0)
    si = pl.program_id(1)
    rows = pl.ds(si * BM, BM)
    x = x_ref[rows, :].astype(jnp.bfloat16)
    w = w_ref[...].astype(jnp.bfloat16)
    t = jnp.dot(x, w, preferred_element_type=jnp.float32)
    out_ref[...] = t

    cols = v * BV + jax.lax.broadcasted_iota(jnp.int32, (BM, BV), 1)
    valid = cols < V
    tm = jnp.where(valid, t, -1e30)

    first = v == 0
    m_old = jnp.where(first, jnp.full((BM, 1), -1e30, jnp.float32),
                      m_scr[rows, :])
    s_old = jnp.where(first, jnp.zeros((BM, 1), jnp.float32), s_scr[rows, :])
    ll_old = jnp.where(first, jnp.zeros((BM, 1), jnp.float32), ll_scr[rows, :])

    m_new = jnp.maximum(m_old, jnp.max(tm, axis=1, keepdims=True))
    s_new = s_old * jnp.exp(m_old - m_new) + jnp.sum(
        jnp.exp(tm - m_new), axis=1, keepdims=True)
    lbl = lbl_ref[...]
    match = (cols == lbl) & valid
    ll_new = ll_old + jnp.sum(jnp.where(match, t, 0.0), axis=1, keepdims=True)

    m_scr[rows, :] = m_new
    s_scr[rows, :] = s_new
    ll_scr[rows, :] = ll_new

    @pl.when(v == NV - 1)
    def _():
        nll = m_new + jnp.log(s_new) - ll_new
        rg = si * BM + jax.lax.broadcasted_iota(jnp.int32, (BM, 1), 0)
        blk = jnp.sum(jnp.where(rg < S - 1, nll, 0.0))
        prev = jnp.where(si == 0, 0.0, acc_scr[0, 0])
        tot = prev + blk
        acc_scr[0, 0] = tot

        @pl.when(si == S // BM - 1)
        def _():
            loss_ref[...] = (tot / (S - 1.0)).reshape(1, 1)


def _decoder_loss(x, w_dec, labels):
    return pl.pallas_call(
        _dec_body,
        grid=(NV, S // BM),
        in_specs=[
            pl.BlockSpec((S, D), lambda v, s: (0, 0)),
            pl.BlockSpec((D, BV), lambda v, s: (0, v)),
            pl.BlockSpec((BM, 1), lambda v, s: (s, 0)),
        ],
        out_specs=[
            pl.BlockSpec((BM, BV), lambda v, s: (s, v)),
            pl.BlockSpec((1, 1), lambda v, s: (0, 0)),
        ],
        out_shape=[
            jax.ShapeDtypeStruct((S, V), jnp.float32),
            jax.ShapeDtypeStruct((1, 1), jnp.float32),
        ],
        scratch_shapes=[
            pltpu.VMEM((S, 1), jnp.float32),
            pltpu.VMEM((S, 1), jnp.float32),
            pltpu.VMEM((S, 1), jnp.float32),
            pltpu.SMEM((1, 1), jnp.float32),
        ],
    )(x, w_dec, labels)


def kernel(inputs, active, responses, peer_weights, Wqkv, bqkv, Wo, bo,
           W1, b1, W2, b2, ln1_g, ln1_b, ln2_g, ln2_b, W_dec):
    noise = jax.random.normal(jax.random.key(42), peer_weights.shape,
                              dtype=jnp.float32)
    af = active.astype(jnp.float32)
    jw = _routing(peer_weights, af, noise)
    x = _combine(jw, responses.reshape(TOPK, S, D))
    for l in range(L):
        q, k, v = _mm_qkv(x, Wqkv[l], bqkv[l])
        o = _attention(q, k, v)
        x = _mm_res_ln(o, Wo[l], bo[l], x, ln1_g[l], ln1_b[l])
        h = _mm_bias(x, W1[l], b1[l], relu=True)
        x = _mm_res_ln(h, W2[l], b2[l], x, ln2_g[l], ln2_b[l])
    labels = jnp.concatenate(
        [inputs[0, 1:], jnp.zeros((1,), inputs.dtype)]).astype(jnp.int32)
    logits, loss = _decoder_loss(x, W_dec, labels.reshape(S, 1))
    return loss.reshape(()), logits.reshape(B, S, V)
